# Initial kernel scaffold; baseline (speedup 1.0000x reference)
import functools

import jax
import jax.numpy as jnp
from jax import lax
from jax.experimental import pallas as pl
from jax.experimental.pallas import tpu as pltpu
from jax.experimental.pallas import tpu_sc as plsc

N = 10000
E = 320000
NC = 2
NS = 16
NW = NC * NS
EPW = E // NW
NPW = N // NS
EPS = 1e-5
F32 = jnp.float32


def _mesh():
    return plsc.VectorSubcoreMesh(core_axis_name="c", subcore_axis_name="s",
                                  num_cores=NC, num_subcores=NS)


def _gather_chunk_size(c):
    return 200 if c > 64 else 400


@functools.partial(jax.jit, static_argnames=("c",))
def _edge_gather(P, Q, src, dst, c):
    K = _gather_chunk_size(c)
    nch = EPW // K

    def body(p_hbm, q_hbm, src_hbm, dst_hbm, y_hbm, st_hbm,
             idx_s, idx_d, pg, qg, yb, accb, sem1, sem2):
        wid = lax.axis_index("s") * NC + lax.axis_index("c")
        base_w = wid * EPW
        for t in range(2 * (c // 16)):
            accb[t // (c // 16), pl.ds((t % (c // 16)) * 16, 16)] = (
                jnp.zeros((16,), F32))

        def chunk(i, carry):
            base = base_w + i * K
            pltpu.sync_copy(dst_hbm.at[pl.ds(base, K)], idx_d)
            pltpu.sync_copy(src_hbm.at[pl.ds(base, K)], idx_s)
            cp1 = pltpu.async_copy(p_hbm.at[idx_d], pg, sem1)
            cp2 = pltpu.async_copy(q_hbm.at[idx_s], qg, sem2)
            cp1.wait()
            cp2.wait()
            for s in range(c // 16):
                sl = pl.ds(s * 16, 16)

                def row(j, acc):
                    a, a2 = acc
                    y = pg[j, sl] + qg[j, sl]
                    yb[j, sl] = y
                    return (a + y, a2 + y * y)

                a, a2 = lax.fori_loop(
                    0, K, row, (jnp.zeros((16,), F32), jnp.zeros((16,), F32)))
                accb[0, sl] += a
                accb[1, sl] += a2
            pltpu.sync_copy(yb, y_hbm.at[pl.ds(base, K)])
            return carry

        lax.fori_loop(0, nch, chunk, 0)
        pltpu.sync_copy(accb, st_hbm.at[pl.ds(wid * 2, 2)])

    return pl.kernel(
        body,
        out_type=[jax.ShapeDtypeStruct((E, c), F32),
                  jax.ShapeDtypeStruct((2 * NW, c), F32)],
        mesh=_mesh(),
        scratch_types=[
            pltpu.VMEM((K,), jnp.int32), pltpu.VMEM((K,), jnp.int32),
            pltpu.VMEM((K, c), F32), pltpu.VMEM((K, c), F32),
            pltpu.VMEM((K, c), F32), pltpu.VMEM((2, c), F32),
            pltpu.SemaphoreType.DMA, pltpu.SemaphoreType.DMA,
        ],
    )(P, Q, src, dst)


@functools.partial(jax.jit, static_argnames=("c", "with_ones"))
def _edge_scatter(y2, dst, ab, c, with_ones):
    W = c + 16 if with_ones else c
    K = _gather_chunk_size(c)
    nch = EPW // K
    RB = 125

    def body(y_hbm, dst_hbm, ab_hbm, agg_hbm,
             idx_d, yb, hb, tb, abv, agg_sp):
        cid = lax.axis_index("c")
        sid = lax.axis_index("s")
        wid = sid * NC + cid
        pltpu.sync_copy(ab_hbm, abv)
        for s in range(W // 16):
            sl = pl.ds(s * 16, 16)

            def zrow(j, carry):
                tb[j, sl] = jnp.zeros((16,), F32)
                return carry

            lax.fori_loop(0, RB, zrow, 0)
        if with_ones:
            sl1 = pl.ds(c, 16)

            def onerow(j, carry):
                hb[j, sl1] = jnp.ones((16,), F32)
                return carry

            lax.fori_loop(0, K, onerow, 0)
        for r in range(NPW // RB):
            pltpu.sync_copy(tb, agg_sp.at[pl.ds(sid * NPW + r * RB, RB)])
        plsc.subcore_barrier()

        base_w = wid * EPW

        def chunk(i, carry):
            base = base_w + i * K
            pltpu.sync_copy(dst_hbm.at[pl.ds(base, K)], idx_d)
            pltpu.sync_copy(y_hbm.at[pl.ds(base, K)], yb)
            for s in range(c // 16):
                sl = pl.ds(s * 16, 16)
                av = abv[0, sl]
                bv = abv[1, sl]

                def row(j, cr):
                    hb[j, sl] = jnp.maximum(yb[j, sl] * av + bv, 0.0)
                    return cr

                lax.fori_loop(0, K, row, 0)
            pltpu.sync_copy(hb, agg_sp.at[idx_d], add=True)
            return carry

        lax.fori_loop(0, nch, chunk, 0)
        plsc.subcore_barrier()
        for r in range(NPW // RB):
            row0 = sid * NPW + r * RB
            pltpu.sync_copy(agg_sp.at[pl.ds(row0, RB)], tb)
            pltpu.sync_copy(tb, agg_hbm.at[pl.ds(cid * N + row0, RB)])

    return pl.kernel(
        body,
        out_type=jax.ShapeDtypeStruct((2 * N, W), F32),
        mesh=_mesh(),
        scratch_types=[
            pltpu.VMEM((K,), jnp.int32),
            pltpu.VMEM((K, c), F32), pltpu.VMEM((K, W), F32),
            pltpu.VMEM((RB, W), F32), pltpu.VMEM((2, c), F32),
            pltpu.VMEM_SHARED((N, W), F32),
        ],
    )(y2, dst, ab)


T_EDGE = 2500


@functools.partial(jax.jit, static_argnames=("c1", "c2"))
def _edge_matmul(y1, stats, g1, b1, W2, g2, b2, c1, c2):
    grid = E // T_EDGE

    def body(y_ref, st_ref, g1_ref, b1_ref, w_ref, g2_ref, b2_ref,
             y2_ref, ab_ref, acc):
        i = pl.program_id(0)
        st = st_ref[...].reshape(NW, 2, c1)
        ssum = jnp.sum(st[:, 0, :], axis=0)
        ssq = jnp.sum(st[:, 1, :], axis=0)
        mu = ssum / E
        var = ssq / E - mu * mu
        a1 = g1_ref[...] / jnp.sqrt(var[None, :] + EPS)
        b1p = b1_ref[...] - mu[None, :] * a1
        h = jnp.maximum(y_ref[...] * a1 + b1p, 0.0)
        y2 = jnp.dot(h, w_ref[...], preferred_element_type=F32)
        y2_ref[...] = y2
        part = jnp.stack([jnp.sum(y2, axis=0), jnp.sum(y2 * y2, axis=0)])
        acc[...] = jnp.where(i == 0, part, acc[...] + part)

        @pl.when(i == grid - 1)
        def _():
            mu2 = acc[0, :] / E
            var2 = acc[1, :] / E - mu2 * mu2
            a2 = g2_ref[0, :] / jnp.sqrt(var2 + EPS)
            b2p = b2_ref[0, :] - mu2 * a2
            ab_ref[...] = jnp.stack([a2, b2p])

    return pl.pallas_call(
        body,
        grid=(grid,),
        in_specs=[
            pl.BlockSpec((T_EDGE, c1), lambda i: (i, 0)),
            pl.BlockSpec((2 * NW, c1), lambda i: (0, 0)),
            pl.BlockSpec((1, c1), lambda i: (0, 0)),
            pl.BlockSpec((1, c1), lambda i: (0, 0)),
            pl.BlockSpec((c1, c2), lambda i: (0, 0)),
            pl.BlockSpec((1, c2), lambda i: (0, 0)),
            pl.BlockSpec((1, c2), lambda i: (0, 0)),
        ],
        out_specs=[
            pl.BlockSpec((T_EDGE, c2), lambda i: (i, 0)),
            pl.BlockSpec((2, c2), lambda i: (0, 0)),
        ],
        out_shape=[jax.ShapeDtypeStruct((E, c2), F32),
                   jax.ShapeDtypeStruct((2, c2), F32)],
        scratch_shapes=[pltpu.VMEM((2, c2), F32)],
    )(y1, stats, g1, b1, W2, g2, b2)


def _bn_cols(t, g, b):
    mu = jnp.mean(t, axis=0, keepdims=True)
    var = jnp.mean((t - mu) ** 2, axis=0, keepdims=True)
    return (t - mu) / jnp.sqrt(var + EPS) * g + b


@jax.jit
def _init_node(features, g, b, A, B):
    def body(f_ref, g_ref, b_ref, a_ref, bb_ref, x_ref, p_ref, q_ref):
        xb = _bn_cols(f_ref[...], g_ref[...], b_ref[...])
        x_ref[...] = xb
        p_ref[...] = jnp.dot(xb, a_ref[...], preferred_element_type=F32)
        q_ref[...] = jnp.dot(xb, bb_ref[...], preferred_element_type=F32)

    d = features.shape[1]
    cn = A.shape[1]
    return pl.pallas_call(
        body,
        out_shape=[jax.ShapeDtypeStruct((N, d), F32),
                   jax.ShapeDtypeStruct((N, cn), F32),
                   jax.ShapeDtypeStruct((N, cn), F32)],
    )(features, g, b, A, B)


@functools.partial(jax.jit, static_argnames=("c", "has_sc", "has_next",
                                             "first"))
def _node_update(aggpair, degcol, x, scW, scg, scb, A, B, c, has_sc,
                 has_next, first):
    def body(*refs):
        it = iter(refs)
        ap_ref = next(it)
        deg_ref = None if first else next(it)
        x_ref = next(it)
        if has_sc:
            scw_ref, scg_ref, scb_ref = next(it), next(it), next(it)
        if has_next:
            a_ref, b_ref = next(it), next(it)
        xo_ref = next(it)
        if has_next:
            po_ref, qo_ref = next(it), next(it)
        if first:
            dego_ref = next(it)
        ap = ap_ref[...]
        if first:
            agg = ap[0, :, :c] + ap[1, :, :c]
            deg = jnp.maximum(ap[0, :, c:c + 1] + ap[1, :, c:c + 1], 1.0)
            dego_ref[...] = deg
        else:
            agg = ap[0] + ap[1]
            deg = deg_ref[...]
        agg = agg / deg
        x = x_ref[...]
        if has_sc:
            sc = _bn_cols(
                jnp.dot(x, scw_ref[...], preferred_element_type=F32),
                scg_ref[...], scb_ref[...])
        else:
            sc = x
        xn = jnp.maximum(agg + sc, 0.0)
        xo_ref[...] = xn
        if has_next:
            po_ref[...] = jnp.dot(xn, a_ref[...], preferred_element_type=F32)
            qo_ref[...] = jnp.dot(xn, b_ref[...], preferred_element_type=F32)

    args = [aggpair]
    if not first:
        args.append(degcol)
    args.append(x)
    if has_sc:
        args += [scW, scg, scb]
    if has_next:
        args += [A, B]
    outs = [jax.ShapeDtypeStruct((N, c), F32)]
    if has_next:
        cn = A.shape[1]
        outs += [jax.ShapeDtypeStruct((N, cn), F32),
                 jax.ShapeDtypeStruct((N, cn), F32)]
    if first:
        outs.append(jax.ShapeDtypeStruct((N, 1), F32))
    return pl.pallas_call(body, out_shape=outs)(*args)


@jax.jit
def _head(xs, gids, fus_W, fus_bias, fus_g, fus_b, fc1_W, fc1_b, fc2_W,
          fc2_b):
    nx = len(xs)

    def body(*refs):
        x_refs = refs[:nx]
        (gid_ref, fw_ref, fbias_ref, fg_ref, fb_ref, w1_ref, b1_ref,
         w2_ref, b2_ref, out_ref) = refs[nx:]
        fused = jnp.concatenate([r[...] for r in x_refs], axis=1)
        h = jnp.maximum(
            jnp.dot(fused, fw_ref[...], preferred_element_type=F32)
            + fbias_ref[...], 0.0)
        h = _bn_cols(h, fg_ref[...], fb_ref[...])
        gid = gid_ref[...]
        onehot = (gid == lax.broadcasted_iota(jnp.int32, (1, 64), 1)
                  ).astype(F32)
        gsum = lax.dot_general(onehot, h, (((0,), (0,)), ((), ())),
                               preferred_element_type=F32)
        gcnt = jnp.maximum(jnp.sum(onehot, axis=0), 1.0).reshape(64, 1)
        gx = gsum / gcnt
        h1 = jnp.maximum(
            jnp.dot(gx, w1_ref[...], preferred_element_type=F32)
            + b1_ref[...], 0.0)
        out_ref[...] = (jnp.dot(h1, w2_ref[...], preferred_element_type=F32)
                        + b2_ref[...])

    nclass = fc2_W.shape[1]
    return pl.pallas_call(
        body,
        out_shape=jax.ShapeDtypeStruct((64, nclass), F32),
    )(*xs, gids, fus_W, fus_bias, fus_g, fus_b, fc1_W, fc1_b, fc2_W, fc2_b)


def kernel(features, edge_index, graph_ids, params):
    src = edge_index[0]
    dst = edge_index[1]
    blocks = params["blocks"]
    nblk = len(blocks)
    AB = []
    feats = [features.shape[1]]
    for blk in blocks:
        f = feats[-1]
        W1 = blk["convs"][0]["W"]
        AB.append((W1[:f] - W1[f:], W1[f:]))
        feats.append(blk["convs"][-1]["W"].shape[1])

    x, P, Q = _init_node(features, params["bn0_g"].reshape(1, -1),
                         params["bn0_b"].reshape(1, -1), AB[0][0], AB[0][1])
    degcol = None
    outputs = []
    for bi, blk in enumerate(blocks):
        c = feats[bi + 1]
        cv1, cv2 = blk["convs"]
        y1, stats = _edge_gather(P, Q, src, dst, c=c)
        y2, ab = _edge_matmul(
            y1, stats, cv1["g"].reshape(1, -1), cv1["b"].reshape(1, -1),
            cv2["W"], cv2["g"].reshape(1, -1), cv2["b"].reshape(1, -1),
            c1=c, c2=c)
        first = bi == 0
        aggflat = _edge_scatter(y2, dst, ab, c=c, with_ones=first)
        aggpair = aggflat.reshape(2, N, aggflat.shape[1])
        has_sc = "sc_W" in blk
        has_next = bi + 1 < nblk
        nxt = AB[bi + 1] if has_next else (None, None)
        res = _node_update(
            aggpair, degcol, x,
            blk.get("sc_W"),
            blk["sc_g"].reshape(1, -1) if has_sc else None,
            blk["sc_b"].reshape(1, -1) if has_sc else None,
            nxt[0], nxt[1], c=c, has_sc=has_sc, has_next=has_next,
            first=first)
        x = res[0]
        if has_next:
            P, Q = res[1], res[2]
        if first:
            degcol = res[-1]
        outputs.append(x)

    return _head(outputs, graph_ids.reshape(N, 1).astype(jnp.int32),
                 params["fus_W"], params["fus_bias"].reshape(1, -1),
                 params["fus_g"].reshape(1, -1),
                 params["fus_b"].reshape(1, -1),
                 params["fc1_W"], params["fc1_b"].reshape(1, -1),
                 params["fc2_W"], params["fc2_b"].reshape(1, -1))


# speedup vs baseline: 2.1017x; 2.1017x over previous
import functools

import jax
import jax.numpy as jnp
from jax import lax
from jax.experimental import pallas as pl
from jax.experimental.pallas import tpu as pltpu
from jax.experimental.pallas import tpu_sc as plsc

N = 10000
E = 320000
NC = 2
NS = 16
NW = NC * NS
EPW = E // NW
NP = 10240
NPW = NP // NS
CP = 128
K = 200
KS = 80
RB = 64
EPS = 1e-5
F32 = jnp.float32


def _mesh():
    return plsc.VectorSubcoreMesh(core_axis_name="c", subcore_axis_name="s",
                                  num_cores=NC, num_subcores=NS)


def _pad_cols(w, width=CP):
    return jnp.pad(w, ((0, 0), (0, width - w.shape[1])))


def _pad_mat(w, width=CP):
    return jnp.pad(w, ((0, width - w.shape[0]), (0, width - w.shape[1])))


def _pad_vec(v, width=CP):
    return jnp.pad(v, (0, width - v.shape[0])).reshape(1, width)


@functools.partial(jax.jit, static_argnames=("c",))
def _edge_gather(P, Q, src, dst, c):
    nch = EPW // K
    ns = c // 16

    def body(p_hbm, q_hbm, src_hbm, dst_hbm, y_hbm, st_hbm,
             idx_s, idx_d, pg, qg, yb, accb, sem1, sem2):
        wid = lax.axis_index("s") * NC + lax.axis_index("c")
        base_w = wid * EPW
        for t in range(8 * (CP // 16)):
            accb[t // (CP // 16), pl.ds((t % (CP // 16)) * 16, 16)] = (
                jnp.zeros((16,), F32))
        for s in range(ns, CP // 16):
            sl = pl.ds(s * 16, 16)

            def zrow(j, carry):
                yb[j, sl] = jnp.zeros((16,), F32)
                return carry

            lax.fori_loop(0, K, zrow, 0)

        def chunk(i, carry):
            base = base_w + i * K
            pltpu.sync_copy(dst_hbm.at[pl.ds(base, K)], idx_d)
            pltpu.sync_copy(src_hbm.at[pl.ds(base, K)], idx_s)
            cp1 = pltpu.async_copy(p_hbm.at[idx_d], pg, sem1)
            cp2 = pltpu.async_copy(q_hbm.at[idx_s], qg, sem2)
            cp1.wait()
            cp2.wait()
            for s in range(ns):
                sl = pl.ds(s * 16, 16)

                def row(j, acc):
                    a, a2 = acc
                    y = pg[j, sl] + qg[j, sl]
                    yb[j, sl] = y
                    return (a + y, a2 + y * y)

                a, a2 = lax.fori_loop(
                    0, K, row, (jnp.zeros((16,), F32), jnp.zeros((16,), F32)))
                accb[0, sl] += a
                accb[1, sl] += a2
            pltpu.sync_copy(yb, y_hbm.at[pl.ds(base, K)])
            return carry

        lax.fori_loop(0, nch, chunk, 0)
        pltpu.sync_copy(accb, st_hbm.at[pl.ds(wid * 8, 8)])

    return pl.kernel(
        body,
        out_type=[jax.ShapeDtypeStruct((E, CP), F32),
                  jax.ShapeDtypeStruct((8 * NW, CP), F32)],
        mesh=_mesh(),
        scratch_types=[
            pltpu.VMEM((K,), jnp.int32), pltpu.VMEM((K,), jnp.int32),
            pltpu.VMEM((K, CP), F32), pltpu.VMEM((K, CP), F32),
            pltpu.VMEM((K, CP), F32), pltpu.VMEM((8, CP), F32),
            pltpu.SemaphoreType.DMA, pltpu.SemaphoreType.DMA,
        ],
    )(P, Q, src, dst)


@functools.partial(jax.jit, static_argnames=("c", "with_ones"))
def _edge_scatter(y2, dst, ab, c, with_ones):
    nch = EPW // KS
    ns = c // 16

    def body(y_hbm, dst_hbm, ab_hbm, agg_hbm,
             idx_d, yb, hb, tb, abv, agg_sp):
        cid = lax.axis_index("c")
        sid = lax.axis_index("s")
        wid = sid * NC + cid
        pltpu.sync_copy(ab_hbm, abv)
        for s in range(CP // 16):
            sl = pl.ds(s * 16, 16)

            def zrow(j, carry):
                tb[j, sl] = jnp.zeros((16,), F32)
                return carry

            lax.fori_loop(0, RB, zrow, 0)
        for s in range(ns, CP // 16):
            sl = pl.ds(s * 16, 16)
            fill = (jnp.ones((16,), F32) if (with_ones and s == ns)
                    else jnp.zeros((16,), F32))

            def prow(j, carry, sl=sl, fill=fill):
                hb[j, sl] = fill
                return carry

            lax.fori_loop(0, KS, prow, 0)
        for r in range(NPW // RB):
            pltpu.sync_copy(tb, agg_sp.at[pl.ds(sid * NPW + r * RB, RB)])
        plsc.subcore_barrier()

        base_w = wid * EPW

        def chunk(i, carry):
            base = base_w + i * KS
            pltpu.sync_copy(dst_hbm.at[pl.ds(base, KS)], idx_d)
            pltpu.sync_copy(y_hbm.at[pl.ds(base, KS)], yb)
            for s in range(ns):
                sl = pl.ds(s * 16, 16)
                av = abv[0, sl]
                bv = abv[1, sl]

                def row(j, cr):
                    hb[j, sl] = jnp.maximum(yb[j, sl] * av + bv, 0.0)
                    return cr

                lax.fori_loop(0, KS, row, 0)
            pltpu.sync_copy(hb, agg_sp.at[idx_d], add=True)
            return carry

        lax.fori_loop(0, nch, chunk, 0)
        plsc.subcore_barrier()
        for r in range(NPW // RB):
            row0 = sid * NPW + r * RB
            pltpu.sync_copy(agg_sp.at[pl.ds(row0, RB)], tb)
            pltpu.sync_copy(tb, agg_hbm.at[pl.ds(cid * NP + row0, RB)])

    return pl.kernel(
        body,
        out_type=jax.ShapeDtypeStruct((2 * NP, CP), F32),
        mesh=_mesh(),
        scratch_types=[
            pltpu.VMEM((KS,), jnp.int32),
            pltpu.VMEM((KS, CP), F32), pltpu.VMEM((KS, CP), F32),
            pltpu.VMEM((RB, CP), F32), pltpu.VMEM((2, CP), F32),
            pltpu.VMEM_SHARED((NP, CP), F32),
        ],
    )(y2, dst, ab)


T_EDGE = 2000


@jax.jit
def _edge_matmul(y1, stats, g1, b1, W2, g2, b2):
    grid = E // T_EDGE

    def body(y_ref, st_ref, g1_ref, b1_ref, w_ref, g2_ref, b2_ref,
             y2_ref, ab_ref, acc):
        i = pl.program_id(0)
        st = st_ref[...].reshape(NW, 8, CP)
        ssum = jnp.sum(st[:, 0, :], axis=0)
        ssq = jnp.sum(st[:, 1, :], axis=0)
        mu = ssum / E
        var = ssq / E - mu * mu
        a1 = g1_ref[...] / jnp.sqrt(var[None, :] + EPS)
        b1p = b1_ref[...] - mu[None, :] * a1
        h = jnp.maximum(y_ref[...] * a1 + b1p, 0.0)
        y2 = jnp.dot(h, w_ref[...], preferred_element_type=F32,
                    precision=lax.Precision.HIGHEST)
        y2_ref[...] = y2
        part = jnp.stack([jnp.sum(y2, axis=0), jnp.sum(y2 * y2, axis=0)])
        acc[...] = jnp.where(i == 0, part, acc[...] + part)

        @pl.when(i == grid - 1)
        def _():
            mu2 = acc[0, :] / E
            var2 = acc[1, :] / E - mu2 * mu2
            a2 = g2_ref[0, :] / jnp.sqrt(var2 + EPS)
            b2p = b2_ref[0, :] - mu2 * a2
            ab_ref[...] = jnp.stack([a2, b2p])

    return pl.pallas_call(
        body,
        grid=(grid,),
        in_specs=[
            pl.BlockSpec((T_EDGE, CP), lambda i: (i, 0)),
            pl.BlockSpec((8 * NW, CP), lambda i: (0, 0)),
            pl.BlockSpec((1, CP), lambda i: (0, 0)),
            pl.BlockSpec((1, CP), lambda i: (0, 0)),
            pl.BlockSpec((CP, CP), lambda i: (0, 0)),
            pl.BlockSpec((1, CP), lambda i: (0, 0)),
            pl.BlockSpec((1, CP), lambda i: (0, 0)),
        ],
        out_specs=[
            pl.BlockSpec((T_EDGE, CP), lambda i: (i, 0)),
            pl.BlockSpec((2, CP), lambda i: (0, 0)),
        ],
        out_shape=[jax.ShapeDtypeStruct((E, CP), F32),
                   jax.ShapeDtypeStruct((2, CP), F32)],
        scratch_shapes=[pltpu.VMEM((2, CP), F32)],
    )(y1, stats, g1, b1, W2, g2, b2)


def _bn_cols(t, g, b):
    mu = jnp.mean(t, axis=0, keepdims=True)
    var = jnp.mean((t - mu) ** 2, axis=0, keepdims=True)
    return (t - mu) / jnp.sqrt(var + EPS) * g + b


@jax.jit
def _init_node(features, g, b):
    def body(f_ref, g_ref, b_ref, x_ref):
        x_ref[...] = _bn_cols(f_ref[...], g_ref[...], b_ref[...])

    d = features.shape[1]
    return pl.pallas_call(
        body,
        out_shape=jax.ShapeDtypeStruct((N, d), F32),
    )(features, g, b)


T_PQ = 2000


@jax.jit
def _pq(x, A, B):
    c = x.shape[1]

    def body(x_ref, a_ref, b_ref, p_ref, q_ref):
        xv = x_ref[...]
        p_ref[...] = jnp.dot(xv, a_ref[...], preferred_element_type=F32,
                             precision=lax.Precision.HIGHEST)
        q_ref[...] = jnp.dot(xv, b_ref[...], preferred_element_type=F32,
                             precision=lax.Precision.HIGHEST)

    return pl.pallas_call(
        body,
        grid=(N // T_PQ,),
        in_specs=[
            pl.BlockSpec((T_PQ, c), lambda i: (i, 0)),
            pl.BlockSpec((c, CP), lambda i: (0, 0)),
            pl.BlockSpec((c, CP), lambda i: (0, 0)),
        ],
        out_specs=[
            pl.BlockSpec((T_PQ, CP), lambda i: (i, 0)),
            pl.BlockSpec((T_PQ, CP), lambda i: (i, 0)),
        ],
        out_shape=[jax.ShapeDtypeStruct((N, CP), F32),
                   jax.ShapeDtypeStruct((N, CP), F32)],
    )(x, A, B)


@functools.partial(jax.jit, static_argnames=("c", "has_sc", "first"))
def _node_update(aggpair, degcol, x, scW, scg, scb, c, has_sc,
                 first):
    def body(*refs):
        it = iter(refs)
        ap_ref = next(it)
        deg_ref = None if first else next(it)
        x_ref = next(it)
        if has_sc:
            scw_ref, scg_ref, scb_ref = next(it), next(it), next(it)
        xo_ref = next(it)
        if first:
            dego_ref = next(it)
        ap = ap_ref[...]
        agg = ap[0, :N, :c] + ap[1, :N, :c]
        if first:
            deg = jnp.maximum(ap[0, :N, c:c + 1] + ap[1, :N, c:c + 1], 1.0)
            dego_ref[...] = deg
        else:
            deg = deg_ref[...]
        agg = agg / deg
        x = x_ref[...]
        if has_sc:
            sc = _bn_cols(
                jnp.dot(x, scw_ref[...], preferred_element_type=F32,
                    precision=lax.Precision.HIGHEST),
                scg_ref[...], scb_ref[...])[:, :c]
        else:
            sc = x
        xn = jnp.maximum(agg + sc, 0.0)
        xo_ref[...] = xn

    args = [aggpair]
    if not first:
        args.append(degcol)
    args.append(x)
    if has_sc:
        args += [scW, scg, scb]
    outs = [jax.ShapeDtypeStruct((N, c), F32)]
    if first:
        outs.append(jax.ShapeDtypeStruct((N, 1), F32))
    return pl.pallas_call(body, out_shape=outs)(*args)


@jax.jit
def _head_fuse(xs, fws, fus_bias):
    nx = len(xs)
    FO = fus_bias.shape[1]
    grid = N // T_PQ

    def body(*refs):
        x_refs = refs[:nx]
        fw_refs = refs[nx:2 * nx]
        fbias_ref = refs[2 * nx]
        h_ref, st_ref, acc = refs[2 * nx + 1:]
        i = pl.program_id(0)
        a = fbias_ref[...]
        for xr, wr in zip(x_refs, fw_refs):
            a = a + jnp.dot(xr[...], wr[...], preferred_element_type=F32,
                            precision=lax.Precision.HIGHEST)
        h = jnp.maximum(a, 0.0)
        h_ref[...] = h
        part = jnp.stack([jnp.sum(h, axis=0), jnp.sum(h * h, axis=0)])
        acc[...] = jnp.where(i == 0, part, acc[...] + part)

        @pl.when(i == grid - 1)
        def _():
            st_ref[...] = acc[...]

    cs = [x.shape[1] for x in xs]
    return pl.pallas_call(
        body,
        grid=(grid,),
        in_specs=(
            [pl.BlockSpec((T_PQ, c), lambda i: (i, 0)) for c in cs]
            + [pl.BlockSpec((c, FO), lambda i: (0, 0)) for c in cs]
            + [pl.BlockSpec((1, FO), lambda i: (0, 0))]),
        out_specs=[
            pl.BlockSpec((T_PQ, FO), lambda i: (i, 0)),
            pl.BlockSpec((2, FO), lambda i: (0, 0)),
        ],
        out_shape=[jax.ShapeDtypeStruct((N, FO), F32),
                   jax.ShapeDtypeStruct((2, FO), F32)],
        scratch_shapes=[pltpu.VMEM((2, FO), F32)],
    )(*xs, *fws, fus_bias)


@functools.partial(jax.jit, static_argnames=("nclass_in",))
def _head_out(hraw, st, gids, fus_g, fus_b, fc1_W, fc1_b, fc2_W, fc2_b,
              nclass_in):
    def body(h_ref, st_ref, gid_ref, fg_ref, fb_ref, w1_ref, b1_ref,
             w2_ref, b2_ref, out_ref):
        stv = st_ref[...]
        mu = stv[0:1, :] / N
        var = stv[1:2, :] / N - mu * mu
        ga = fg_ref[...] / jnp.sqrt(var + EPS)
        gb = fb_ref[...] - mu * ga
        gid = gid_ref[...]
        onehot = (gid == lax.broadcasted_iota(jnp.int32, (1, 64), 1)
                  ).astype(F32)
        gsum = lax.dot_general(onehot, h_ref[...], (((0,), (0,)), ((), ())),
                               preferred_element_type=F32,
                               precision=lax.Precision.HIGHEST)
        gcnt = jnp.maximum(jnp.sum(onehot, axis=0), 1.0).reshape(64, 1)
        gx = (gsum / gcnt) * ga + gb
        h1 = jnp.maximum(
            jnp.dot(gx, w1_ref[...], preferred_element_type=F32,
                    precision=lax.Precision.HIGHEST)
            + b1_ref[...], 0.0)
        out_ref[...] = (jnp.dot(h1, w2_ref[...], preferred_element_type=F32,
                    precision=lax.Precision.HIGHEST)
                        + b2_ref[...])[:, :out_ref.shape[1]]

    return pl.pallas_call(
        body,
        out_shape=jax.ShapeDtypeStruct((64, nclass_in), F32),
    )(hraw, st, gids, fus_g, fus_b, fc1_W, fc1_b, fc2_W, fc2_b)


def kernel(features, edge_index, graph_ids, params):
    src = edge_index[0]
    dst = edge_index[1]
    blocks = params["blocks"]
    nblk = len(blocks)
    AB = []
    feats = [features.shape[1]]
    for blk in blocks:
        f = feats[-1]
        W1 = blk["convs"][0]["W"]
        AB.append((_pad_cols(W1[:f] - W1[f:]), _pad_cols(W1[f:])))
        feats.append(blk["convs"][-1]["W"].shape[1])

    x = _init_node(features, params["bn0_g"].reshape(1, -1),
                   params["bn0_b"].reshape(1, -1))
    P, Q = _pq(x, AB[0][0], AB[0][1])
    degcol = None
    outputs = []
    for bi, blk in enumerate(blocks):
        c = feats[bi + 1]
        cv1, cv2 = blk["convs"]
        y1, stats = _edge_gather(P, Q, src, dst, c=c)
        y2, ab = _edge_matmul(
            y1, stats, _pad_vec(cv1["g"]), _pad_vec(cv1["b"]),
            _pad_mat(cv2["W"]), _pad_vec(cv2["g"]), _pad_vec(cv2["b"]))
        first = bi == 0
        aggflat = _edge_scatter(y2, dst, ab, c=c, with_ones=first)
        aggpair = aggflat.reshape(2, NP, CP)
        has_sc = "sc_W" in blk
        res = _node_update(
            aggpair, degcol, x,
            _pad_cols(blk["sc_W"]) if has_sc else None,
            _pad_vec(blk["sc_g"]) if has_sc else None,
            _pad_vec(blk["sc_b"]) if has_sc else None,
            c=c, has_sc=has_sc, first=first)
        x = res[0]
        if first:
            degcol = res[-1]
        if bi + 1 < nblk:
            P, Q = _pq(x, AB[bi + 1][0], AB[bi + 1][1])
        outputs.append(x)

    fw = params["fus_W"]
    fws, off = [], 0
    for bi in range(nblk):
        fws.append(fw[off:off + feats[bi + 1]])
        off += feats[bi + 1]
    hraw, st = _head_fuse(outputs, fws, params["fus_bias"].reshape(1, -1))
    return _head_out(hraw, st, graph_ids.reshape(N, 1).astype(jnp.int32),
                     params["fus_g"].reshape(1, -1),
                     params["fus_b"].reshape(1, -1),
                     params["fc1_W"], params["fc1_b"].reshape(1, -1),
                     _pad_cols(params["fc2_W"]), _pad_vec(params["fc2_b"]),
                     nclass_in=params["fc2_W"].shape[1])


# speedup vs baseline: 3.6291x; 1.7268x over previous
import functools

import jax
import jax.numpy as jnp
from jax import lax
from jax.experimental import pallas as pl
from jax.experimental.pallas import tpu as pltpu
from jax.experimental.pallas import tpu_sc as plsc

N = 10000
E = 320000
NC = 2
NS = 16
NW = NC * NS
EPW = E // NW
NP = 10240
NPW = NP // NS
CP = 128
K = 200
KS = 80
RB = 128
EPS = 1e-5
F32 = jnp.float32


def _mesh():
    return plsc.VectorSubcoreMesh(core_axis_name="c", subcore_axis_name="s",
                                  num_cores=NC, num_subcores=NS)


def _pad_cols(w, width=CP):
    return jnp.pad(w, ((0, 0), (0, width - w.shape[1])))


def _pad_mat(w, width=CP):
    return jnp.pad(w, ((0, width - w.shape[0]), (0, width - w.shape[1])))


def _pad_vec(v, width=CP):
    return jnp.pad(v, (0, width - v.shape[0])).reshape(1, width)


@functools.partial(jax.jit, static_argnames=("c",))
def _edge_gather(P, Q, src, dst, c):
    nch = EPW // K
    ns = c // 16

    def body(p_hbm, q_hbm, src_hbm, dst_hbm, y_hbm, st_hbm, *scr):
        (idx_s0, idx_s1, idx_d0, idx_d1, pg0, pg1, qg0, qg1, accb,
         gsp0, gsp1, gsq0, gsq1, os0, os1) = scr
        idx_s = (idx_s0, idx_s1)
        idx_d = (idx_d0, idx_d1)
        pg = (pg0, pg1)
        qg = (qg0, qg1)
        gsp = (gsp0, gsp1)
        gsq = (gsq0, gsq1)
        osm = (os0, os1)
        wid = lax.axis_index("s") * NC + lax.axis_index("c")
        base_w = wid * EPW
        for t in range(8 * (CP // 16)):
            accb[t // (CP // 16), pl.ds((t % (CP // 16)) * 16, 16)] = (
                jnp.zeros((16,), F32))
        pltpu.sync_copy(dst_hbm.at[pl.ds(base_w, K)], idx_d[0])
        pltpu.sync_copy(src_hbm.at[pl.ds(base_w, K)], idx_s[0])
        pltpu.async_copy(p_hbm.at[idx_d[0]], pg[0], gsp[0])
        pltpu.async_copy(q_hbm.at[idx_s[0]], qg[0], gsq[0])

        def step(i, b):
            nb = 1 - b

            @pl.when(i + 1 < nch)
            def _():
                @pl.when(i >= 1)
                def _():
                    pltpu.make_async_copy(
                        pg[nb], y_hbm.at[pl.ds(base_w, K)], osm[nb]).wait()
                base_n = base_w + (i + 1) * K
                pltpu.sync_copy(dst_hbm.at[pl.ds(base_n, K)], idx_d[nb])
                pltpu.sync_copy(src_hbm.at[pl.ds(base_n, K)], idx_s[nb])
                pltpu.async_copy(p_hbm.at[idx_d[nb]], pg[nb], gsp[nb])
                pltpu.async_copy(q_hbm.at[idx_s[nb]], qg[nb], gsq[nb])

            pltpu.make_async_copy(p_hbm.at[idx_d[b]], pg[b], gsp[b]).wait()
            pltpu.make_async_copy(q_hbm.at[idx_s[b]], qg[b], gsq[b]).wait()
            pgb = pg[b]
            qgb = qg[b]

            def row(j, acc):
                out = []
                for t in range(ns):
                    sl = pl.ds(t * 16, 16)
                    y = pgb[j, sl] + qgb[j, sl]
                    pgb[j, sl] = y
                    out.append(acc[2 * t] + y)
                    out.append(acc[2 * t + 1] + y * y)
                return tuple(out)

            acc = lax.fori_loop(
                0, K, row,
                tuple(jnp.zeros((16,), F32) for _ in range(2 * ns)))
            for t in range(ns):
                sl = pl.ds(t * 16, 16)
                accb[0, sl] += acc[2 * t]
                accb[1, sl] += acc[2 * t + 1]
            pltpu.async_copy(pgb, y_hbm.at[pl.ds(base_w + i * K, K)], osm[b])

        def outer(k, carry):
            step(2 * k, 0)
            step(2 * k + 1, 1)
            return carry

        lax.fori_loop(0, nch // 2, outer, 0)
        pltpu.make_async_copy(
            pg[0], y_hbm.at[pl.ds(base_w, K)], osm[0]).wait()
        pltpu.make_async_copy(
            pg[1], y_hbm.at[pl.ds(base_w, K)], osm[1]).wait()
        pltpu.sync_copy(accb, st_hbm.at[pl.ds(wid * 8, 8)])

    return pl.kernel(
        body,
        out_type=[jax.ShapeDtypeStruct((E, CP), F32),
                  jax.ShapeDtypeStruct((8 * NW, CP), F32)],
        mesh=_mesh(),
        scratch_types=[
            pltpu.VMEM((K,), jnp.int32), pltpu.VMEM((K,), jnp.int32),
            pltpu.VMEM((K,), jnp.int32), pltpu.VMEM((K,), jnp.int32),
            pltpu.VMEM((K, CP), F32), pltpu.VMEM((K, CP), F32),
            pltpu.VMEM((K, CP), F32), pltpu.VMEM((K, CP), F32),
            pltpu.VMEM((8, CP), F32),
            pltpu.SemaphoreType.DMA, pltpu.SemaphoreType.DMA,
            pltpu.SemaphoreType.DMA, pltpu.SemaphoreType.DMA,
            pltpu.SemaphoreType.DMA, pltpu.SemaphoreType.DMA,
        ],
    )(P, Q, src, dst)


@functools.partial(jax.jit, static_argnames=("c", "with_ones"))
def _edge_scatter(y2, dst, ab, c, with_ones):
    nch = EPW // KS
    ns = c // 16

    def body(y_hbm, dst_hbm, ab_hbm, agg_hbm, *scr):
        (idx0, idx1, yb0, yb1, tb, abv, agg_sp, ls0, ls1, ss0, ss1) = scr
        idx = (idx0, idx1)
        yb = (yb0, yb1)
        lsm = (ls0, ls1)
        ssm = (ss0, ss1)
        cid = lax.axis_index("c")
        sid = lax.axis_index("s")
        wid = sid * NC + cid
        pltpu.sync_copy(ab_hbm, abv)
        for t in range(CP // 16):
            sl = pl.ds(t * 16, 16)

            def zrow(j, carry):
                tb[j, sl] = jnp.zeros((16,), F32)
                return carry

            lax.fori_loop(0, RB, zrow, 0)
        for r in range(NPW // RB):
            pltpu.sync_copy(tb, agg_sp.at[pl.ds(sid * NPW + r * RB, RB)])
        plsc.subcore_barrier()

        base_w = wid * EPW
        pltpu.sync_copy(dst_hbm.at[pl.ds(base_w, KS)], idx[0])
        pltpu.async_copy(y_hbm.at[pl.ds(base_w, KS)], yb[0], lsm[0])

        def step(i, b):
            nb = 1 - b

            @pl.when(i < nch)
            def _():
                @pl.when(i + 1 < nch)
                def _():
                    @pl.when(i >= 1)
                    def _():
                        pltpu.make_async_copy(
                            yb[nb], agg_sp.at[idx[nb]], ssm[nb]).wait()
                    base_n = base_w + (i + 1) * KS
                    pltpu.sync_copy(dst_hbm.at[pl.ds(base_n, KS)], idx[nb])
                    pltpu.async_copy(
                        y_hbm.at[pl.ds(base_n, KS)], yb[nb], lsm[nb])

                pltpu.make_async_copy(
                    y_hbm.at[pl.ds(base_w, KS)], yb[b], lsm[b]).wait()
                ybb = yb[b]
                avs = [abv[0, pl.ds(t * 16, 16)] for t in range(ns)]
                bvs = [abv[1, pl.ds(t * 16, 16)] for t in range(ns)]

                def row(j, cr):
                    for t in range(ns):
                        sl = pl.ds(t * 16, 16)
                        ybb[j, sl] = jnp.maximum(
                            ybb[j, sl] * avs[t] + bvs[t], 0.0)
                    if with_ones:
                        ybb[j, pl.ds(c, 16)] = jnp.ones((16,), F32)
                    return cr

                lax.fori_loop(0, KS, row, 0)
                pltpu.async_copy(ybb, agg_sp.at[idx[b]], ssm[b], add=True)

        def outer(k, carry):
            step(2 * k, 0)
            step(2 * k + 1, 1)
            return carry

        lax.fori_loop(0, (nch + 1) // 2, outer, 0)
        pltpu.make_async_copy(yb[0], agg_sp.at[idx[0]], ssm[0]).wait()
        pltpu.make_async_copy(yb[1], agg_sp.at[idx[1]], ssm[1]).wait()
        plsc.subcore_barrier()
        for r in range(NPW // RB):
            row0 = sid * NPW + r * RB
            pltpu.sync_copy(agg_sp.at[pl.ds(row0, RB)], tb)
            pltpu.sync_copy(tb, agg_hbm.at[pl.ds(cid * NP + row0, RB)])

    return pl.kernel(
        body,
        out_type=jax.ShapeDtypeStruct((2 * NP, CP), F32),
        mesh=_mesh(),
        scratch_types=[
            pltpu.VMEM((KS,), jnp.int32), pltpu.VMEM((KS,), jnp.int32),
            pltpu.VMEM((KS, CP), F32), pltpu.VMEM((KS, CP), F32),
            pltpu.VMEM((RB, CP), F32), pltpu.VMEM((2, CP), F32),
            pltpu.VMEM_SHARED((NP, CP), F32),
            pltpu.SemaphoreType.DMA, pltpu.SemaphoreType.DMA,
            pltpu.SemaphoreType.DMA, pltpu.SemaphoreType.DMA,
        ],
    )(y2, dst, ab)


T_EDGE = 2000


@jax.jit
def _edge_matmul(y1, stats, g1, b1, W2, g2, b2):
    grid = E // T_EDGE

    def body(y_ref, st_ref, g1_ref, b1_ref, w_ref, g2_ref, b2_ref,
             y2_ref, ab_ref, acc):
        i = pl.program_id(0)
        st = st_ref[...].reshape(NW, 8, CP)
        ssum = jnp.sum(st[:, 0, :], axis=0)
        ssq = jnp.sum(st[:, 1, :], axis=0)
        mu = ssum / E
        var = ssq / E - mu * mu
        a1 = g1_ref[...] / jnp.sqrt(var[None, :] + EPS)
        b1p = b1_ref[...] - mu[None, :] * a1
        h = jnp.maximum(y_ref[...] * a1 + b1p, 0.0)
        y2 = jnp.dot(h, w_ref[...], preferred_element_type=F32,
                    precision=lax.Precision.HIGHEST)
        y2_ref[...] = y2
        part = jnp.stack([jnp.sum(y2, axis=0), jnp.sum(y2 * y2, axis=0)])
        acc[...] = jnp.where(i == 0, part, acc[...] + part)

        @pl.when(i == grid - 1)
        def _():
            mu2 = acc[0, :] / E
            var2 = acc[1, :] / E - mu2 * mu2
            a2 = g2_ref[0, :] / jnp.sqrt(var2 + EPS)
            b2p = b2_ref[0, :] - mu2 * a2
            ab_ref[...] = jnp.stack([a2, b2p])

    return pl.pallas_call(
        body,
        grid=(grid,),
        in_specs=[
            pl.BlockSpec((T_EDGE, CP), lambda i: (i, 0)),
            pl.BlockSpec((8 * NW, CP), lambda i: (0, 0)),
            pl.BlockSpec((1, CP), lambda i: (0, 0)),
            pl.BlockSpec((1, CP), lambda i: (0, 0)),
            pl.BlockSpec((CP, CP), lambda i: (0, 0)),
            pl.BlockSpec((1, CP), lambda i: (0, 0)),
            pl.BlockSpec((1, CP), lambda i: (0, 0)),
        ],
        out_specs=[
            pl.BlockSpec((T_EDGE, CP), lambda i: (i, 0)),
            pl.BlockSpec((2, CP), lambda i: (0, 0)),
        ],
        out_shape=[jax.ShapeDtypeStruct((E, CP), F32),
                   jax.ShapeDtypeStruct((2, CP), F32)],
        scratch_shapes=[pltpu.VMEM((2, CP), F32)],
    )(y1, stats, g1, b1, W2, g2, b2)


def _bn_cols(t, g, b):
    mu = jnp.mean(t, axis=0, keepdims=True)
    var = jnp.mean((t - mu) ** 2, axis=0, keepdims=True)
    return (t - mu) / jnp.sqrt(var + EPS) * g + b


@jax.jit
def _init_node(features, g, b):
    def body(f_ref, g_ref, b_ref, x_ref):
        x_ref[...] = _bn_cols(f_ref[...], g_ref[...], b_ref[...])

    d = features.shape[1]
    return pl.pallas_call(
        body,
        out_shape=jax.ShapeDtypeStruct((N, d), F32),
    )(features, g, b)


T_PQ = 2000


@jax.jit
def _pq(x, A, B):
    c = x.shape[1]

    def body(x_ref, a_ref, b_ref, p_ref, q_ref):
        xv = x_ref[...]
        p_ref[...] = jnp.dot(xv, a_ref[...], preferred_element_type=F32,
                             precision=lax.Precision.HIGHEST)
        q_ref[...] = jnp.dot(xv, b_ref[...], preferred_element_type=F32,
                             precision=lax.Precision.HIGHEST)

    return pl.pallas_call(
        body,
        grid=(N // T_PQ,),
        in_specs=[
            pl.BlockSpec((T_PQ, c), lambda i: (i, 0)),
            pl.BlockSpec((c, CP), lambda i: (0, 0)),
            pl.BlockSpec((c, CP), lambda i: (0, 0)),
        ],
        out_specs=[
            pl.BlockSpec((T_PQ, CP), lambda i: (i, 0)),
            pl.BlockSpec((T_PQ, CP), lambda i: (i, 0)),
        ],
        out_shape=[jax.ShapeDtypeStruct((N, CP), F32),
                   jax.ShapeDtypeStruct((N, CP), F32)],
    )(x, A, B)


@functools.partial(jax.jit, static_argnames=("c", "has_sc", "first"))
def _node_update(aggpair, degcol, x, scW, scg, scb, c, has_sc,
                 first):
    def body(*refs):
        it = iter(refs)
        ap_ref = next(it)
        deg_ref = None if first else next(it)
        x_ref = next(it)
        if has_sc:
            scw_ref, scg_ref, scb_ref = next(it), next(it), next(it)
        xo_ref = next(it)
        if first:
            dego_ref = next(it)
        ap = ap_ref[...]
        agg = ap[0, :N, :c] + ap[1, :N, :c]
        if first:
            deg = jnp.maximum(ap[0, :N, c:c + 1] + ap[1, :N, c:c + 1], 1.0)
            dego_ref[...] = deg
        else:
            deg = deg_ref[...]
        agg = agg / deg
        x = x_ref[...]
        if has_sc:
            sc = _bn_cols(
                jnp.dot(x, scw_ref[...], preferred_element_type=F32,
                    precision=lax.Precision.HIGHEST),
                scg_ref[...], scb_ref[...])[:, :c]
        else:
            sc = x
        xn = jnp.maximum(agg + sc, 0.0)
        xo_ref[...] = xn

    args = [aggpair]
    if not first:
        args.append(degcol)
    args.append(x)
    if has_sc:
        args += [scW, scg, scb]
    outs = [jax.ShapeDtypeStruct((N, c), F32)]
    if first:
        outs.append(jax.ShapeDtypeStruct((N, 1), F32))
    return pl.pallas_call(body, out_shape=outs)(*args)


@jax.jit
def _head_fuse(xs, fws, fus_bias):
    nx = len(xs)
    FO = fus_bias.shape[1]
    grid = N // T_PQ

    def body(*refs):
        x_refs = refs[:nx]
        fw_refs = refs[nx:2 * nx]
        fbias_ref = refs[2 * nx]
        h_ref, st_ref, acc = refs[2 * nx + 1:]
        i = pl.program_id(0)
        a = fbias_ref[...]
        for xr, wr in zip(x_refs, fw_refs):
            a = a + jnp.dot(xr[...], wr[...], preferred_element_type=F32,
                            precision=lax.Precision.HIGHEST)
        h = jnp.maximum(a, 0.0)
        h_ref[...] = h
        part = jnp.stack([jnp.sum(h, axis=0), jnp.sum(h * h, axis=0)])
        acc[...] = jnp.where(i == 0, part, acc[...] + part)

        @pl.when(i == grid - 1)
        def _():
            st_ref[...] = acc[...]

    cs = [x.shape[1] for x in xs]
    return pl.pallas_call(
        body,
        grid=(grid,),
        in_specs=(
            [pl.BlockSpec((T_PQ, c), lambda i: (i, 0)) for c in cs]
            + [pl.BlockSpec((c, FO), lambda i: (0, 0)) for c in cs]
            + [pl.BlockSpec((1, FO), lambda i: (0, 0))]),
        out_specs=[
            pl.BlockSpec((T_PQ, FO), lambda i: (i, 0)),
            pl.BlockSpec((2, FO), lambda i: (0, 0)),
        ],
        out_shape=[jax.ShapeDtypeStruct((N, FO), F32),
                   jax.ShapeDtypeStruct((2, FO), F32)],
        scratch_shapes=[pltpu.VMEM((2, FO), F32)],
    )(*xs, *fws, fus_bias)


@functools.partial(jax.jit, static_argnames=("nclass_in",))
def _head_out(hraw, st, gids, fus_g, fus_b, fc1_W, fc1_b, fc2_W, fc2_b,
              nclass_in):
    def body(h_ref, st_ref, gid_ref, fg_ref, fb_ref, w1_ref, b1_ref,
             w2_ref, b2_ref, out_ref):
        stv = st_ref[...]
        mu = stv[0:1, :] / N
        var = stv[1:2, :] / N - mu * mu
        ga = fg_ref[...] / jnp.sqrt(var + EPS)
        gb = fb_ref[...] - mu * ga
        gid = gid_ref[...]
        onehot = (gid == lax.broadcasted_iota(jnp.int32, (1, 64), 1)
                  ).astype(F32)
        gsum = lax.dot_general(onehot, h_ref[...], (((0,), (0,)), ((), ())),
                               preferred_element_type=F32,
                               precision=lax.Precision.HIGHEST)
        gcnt = jnp.maximum(jnp.sum(onehot, axis=0), 1.0).reshape(64, 1)
        gx = (gsum / gcnt) * ga + gb
        h1 = jnp.maximum(
            jnp.dot(gx, w1_ref[...], preferred_element_type=F32,
                    precision=lax.Precision.HIGHEST)
            + b1_ref[...], 0.0)
        out_ref[...] = (jnp.dot(h1, w2_ref[...], preferred_element_type=F32,
                    precision=lax.Precision.HIGHEST)
                        + b2_ref[...])[:, :out_ref.shape[1]]

    return pl.pallas_call(
        body,
        out_shape=jax.ShapeDtypeStruct((64, nclass_in), F32),
    )(hraw, st, gids, fus_g, fus_b, fc1_W, fc1_b, fc2_W, fc2_b)


def kernel(features, edge_index, graph_ids, params):
    src = edge_index[0]
    dst = edge_index[1]
    blocks = params["blocks"]
    nblk = len(blocks)
    AB = []
    feats = [features.shape[1]]
    for blk in blocks:
        f = feats[-1]
        W1 = blk["convs"][0]["W"]
        AB.append((_pad_cols(W1[:f] - W1[f:]), _pad_cols(W1[f:])))
        feats.append(blk["convs"][-1]["W"].shape[1])

    x = _init_node(features, params["bn0_g"].reshape(1, -1),
                   params["bn0_b"].reshape(1, -1))
    P, Q = _pq(x, AB[0][0], AB[0][1])
    degcol = None
    outputs = []
    for bi, blk in enumerate(blocks):
        c = feats[bi + 1]
        cv1, cv2 = blk["convs"]
        y1, stats = _edge_gather(P, Q, src, dst, c=c)
        y2, ab = _edge_matmul(
            y1, stats, _pad_vec(cv1["g"]), _pad_vec(cv1["b"]),
            _pad_mat(cv2["W"]), _pad_vec(cv2["g"]), _pad_vec(cv2["b"]))
        first = bi == 0
        aggflat = _edge_scatter(y2, dst, ab, c=c, with_ones=first)
        aggpair = aggflat.reshape(2, NP, CP)
        has_sc = "sc_W" in blk
        res = _node_update(
            aggpair, degcol, x,
            _pad_cols(blk["sc_W"]) if has_sc else None,
            _pad_vec(blk["sc_g"]) if has_sc else None,
            _pad_vec(blk["sc_b"]) if has_sc else None,
            c=c, has_sc=has_sc, first=first)
        x = res[0]
        if first:
            degcol = res[-1]
        if bi + 1 < nblk:
            P, Q = _pq(x, AB[bi + 1][0], AB[bi + 1][1])
        outputs.append(x)

    fw = params["fus_W"]
    fws, off = [], 0
    for bi in range(nblk):
        fws.append(fw[off:off + feats[bi + 1]])
        off += feats[bi + 1]
    hraw, st = _head_fuse(outputs, fws, params["fus_bias"].reshape(1, -1))
    return _head_out(hraw, st, graph_ids.reshape(N, 1).astype(jnp.int32),
                     params["fus_g"].reshape(1, -1),
                     params["fus_b"].reshape(1, -1),
                     params["fc1_W"], params["fc1_b"].reshape(1, -1),
                     _pad_cols(params["fc2_W"]), _pad_vec(params["fc2_b"]),
                     nclass_in=params["fc2_W"].shape[1])


# speedup vs baseline: 3.9837x; 1.0977x over previous
import functools

import jax

jax.config.update("jax_default_matmul_precision", "float32")

import jax.numpy as jnp
from jax import lax
from jax.experimental import pallas as pl
from jax.experimental.pallas import tpu as pltpu
from jax.experimental.pallas import tpu_sc as plsc

N = 10000
E = 320000
NC = 2
NS = 16
NW = NC * NS
EPW = E // NW
NP = 10240
NPW = NP // NS
CP = 128
K = 200
KS = 80
RB = 128
EPS = 1e-5
F32 = jnp.float32


def _mesh():
    return plsc.VectorSubcoreMesh(core_axis_name="c", subcore_axis_name="s",
                                  num_cores=NC, num_subcores=NS)


def _pad_cols(w, width=CP):
    return jnp.pad(w, ((0, 0), (0, width - w.shape[1])))


def _pad_mat(w, width=CP):
    return jnp.pad(w, ((0, width - w.shape[0]), (0, width - w.shape[1])))


def _pad_vec(v, width=CP):
    return jnp.pad(v, (0, width - v.shape[0])).reshape(1, width)


@functools.partial(jax.jit, static_argnames=("c",))
def _edge_gather(P, Q, src, dst, c):
    nch = EPW // K
    ns = c // 16

    def body(p_hbm, q_hbm, src_hbm, dst_hbm, y_hbm, st_hbm, *scr):
        (idx_s0, idx_s1, idx_d0, idx_d1, pg0, pg1, qg0, qg1, accb,
         gsp0, gsp1, gsq0, gsq1, os0, os1) = scr
        idx_s = (idx_s0, idx_s1)
        idx_d = (idx_d0, idx_d1)
        pg = (pg0, pg1)
        qg = (qg0, qg1)
        gsp = (gsp0, gsp1)
        gsq = (gsq0, gsq1)
        osm = (os0, os1)
        wid = lax.axis_index("s") * NC + lax.axis_index("c")
        base_w = wid * EPW
        for t in range(8 * (CP // 16)):
            accb[t // (CP // 16), pl.ds((t % (CP // 16)) * 16, 16)] = (
                jnp.zeros((16,), F32))
        pltpu.sync_copy(dst_hbm.at[pl.ds(base_w, K)], idx_d[0])
        pltpu.sync_copy(src_hbm.at[pl.ds(base_w, K)], idx_s[0])
        pltpu.async_copy(p_hbm.at[idx_d[0]], pg[0], gsp[0])
        pltpu.async_copy(q_hbm.at[idx_s[0]], qg[0], gsq[0])

        def step(i, b):
            nb = 1 - b

            @pl.when(i + 1 < nch)
            def _():
                @pl.when(i >= 1)
                def _():
                    pltpu.make_async_copy(
                        pg[nb], y_hbm.at[pl.ds(base_w, K)], osm[nb]).wait()
                base_n = base_w + (i + 1) * K
                pltpu.sync_copy(dst_hbm.at[pl.ds(base_n, K)], idx_d[nb])
                pltpu.sync_copy(src_hbm.at[pl.ds(base_n, K)], idx_s[nb])
                pltpu.async_copy(p_hbm.at[idx_d[nb]], pg[nb], gsp[nb])
                pltpu.async_copy(q_hbm.at[idx_s[nb]], qg[nb], gsq[nb])

            pltpu.make_async_copy(p_hbm.at[idx_d[b]], pg[b], gsp[b]).wait()
            pltpu.make_async_copy(q_hbm.at[idx_s[b]], qg[b], gsq[b]).wait()
            pgb = pg[b]
            qgb = qg[b]

            def row(j, acc):
                out = []
                for t in range(ns):
                    sl = pl.ds(t * 16, 16)
                    y = pgb[j, sl] + qgb[j, sl]
                    pgb[j, sl] = y
                    out.append(acc[2 * t] + y)
                    out.append(acc[2 * t + 1] + y * y)
                return tuple(out)

            acc = lax.fori_loop(
                0, K, row,
                tuple(jnp.zeros((16,), F32) for _ in range(2 * ns)))
            for t in range(ns):
                sl = pl.ds(t * 16, 16)
                accb[0, sl] += acc[2 * t]
                accb[1, sl] += acc[2 * t + 1]
            pltpu.async_copy(pgb, y_hbm.at[pl.ds(base_w + i * K, K)], osm[b])

        def outer(k, carry):
            step(2 * k, 0)
            step(2 * k + 1, 1)
            return carry

        lax.fori_loop(0, nch // 2, outer, 0)
        pltpu.make_async_copy(
            pg[0], y_hbm.at[pl.ds(base_w, K)], osm[0]).wait()
        pltpu.make_async_copy(
            pg[1], y_hbm.at[pl.ds(base_w, K)], osm[1]).wait()
        pltpu.sync_copy(accb, st_hbm.at[pl.ds(wid * 8, 8)])

    return pl.kernel(
        body,
        out_type=[jax.ShapeDtypeStruct((E, CP), F32),
                  jax.ShapeDtypeStruct((8 * NW, CP), F32)],
        mesh=_mesh(),
        scratch_types=[
            pltpu.VMEM((K,), jnp.int32), pltpu.VMEM((K,), jnp.int32),
            pltpu.VMEM((K,), jnp.int32), pltpu.VMEM((K,), jnp.int32),
            pltpu.VMEM((K, CP), F32), pltpu.VMEM((K, CP), F32),
            pltpu.VMEM((K, CP), F32), pltpu.VMEM((K, CP), F32),
            pltpu.VMEM((8, CP), F32),
            pltpu.SemaphoreType.DMA, pltpu.SemaphoreType.DMA,
            pltpu.SemaphoreType.DMA, pltpu.SemaphoreType.DMA,
            pltpu.SemaphoreType.DMA, pltpu.SemaphoreType.DMA,
        ],
    )(P, Q, src, dst)


@functools.partial(jax.jit, static_argnames=("c", "with_ones"))
def _edge_scatter(y2, dst, ab, c, with_ones):
    nch = EPW // KS
    ns = c // 16

    def body(y_hbm, dst_hbm, ab_hbm, agg_hbm, *scr):
        (idx0, idx1, yb0, yb1, tb, abv, agg_sp, ls0, ls1, ss0, ss1) = scr
        idx = (idx0, idx1)
        yb = (yb0, yb1)
        lsm = (ls0, ls1)
        ssm = (ss0, ss1)
        cid = lax.axis_index("c")
        sid = lax.axis_index("s")
        wid = sid * NC + cid
        pltpu.sync_copy(ab_hbm, abv)
        for t in range(CP // 16):
            sl = pl.ds(t * 16, 16)

            def zrow(j, carry):
                tb[j, sl] = jnp.zeros((16,), F32)
                return carry

            lax.fori_loop(0, RB, zrow, 0)
        for r in range(NPW // RB):
            pltpu.sync_copy(tb, agg_sp.at[pl.ds(sid * NPW + r * RB, RB)])
        plsc.subcore_barrier()

        base_w = wid * EPW
        pltpu.sync_copy(dst_hbm.at[pl.ds(base_w, KS)], idx[0])
        pltpu.async_copy(y_hbm.at[pl.ds(base_w, KS)], yb[0], lsm[0])

        def step(i, b):
            nb = 1 - b

            @pl.when(i < nch)
            def _():
                @pl.when(i + 1 < nch)
                def _():
                    @pl.when(i >= 1)
                    def _():
                        pltpu.make_async_copy(
                            yb[nb], agg_sp.at[idx[nb]], ssm[nb]).wait()
                    base_n = base_w + (i + 1) * KS
                    pltpu.sync_copy(dst_hbm.at[pl.ds(base_n, KS)], idx[nb])
                    pltpu.async_copy(
                        y_hbm.at[pl.ds(base_n, KS)], yb[nb], lsm[nb])

                pltpu.make_async_copy(
                    y_hbm.at[pl.ds(base_w, KS)], yb[b], lsm[b]).wait()
                ybb = yb[b]
                avs = [abv[0, pl.ds(t * 16, 16)] for t in range(ns)]
                bvs = [abv[1, pl.ds(t * 16, 16)] for t in range(ns)]

                def row(j, cr):
                    for t in range(ns):
                        sl = pl.ds(t * 16, 16)
                        ybb[j, sl] = jnp.maximum(
                            ybb[j, sl] * avs[t] + bvs[t], 0.0)
                    if with_ones:
                        ybb[j, pl.ds(c, 16)] = jnp.ones((16,), F32)
                    return cr

                lax.fori_loop(0, KS, row, 0)
                pltpu.async_copy(ybb, agg_sp.at[idx[b]], ssm[b], add=True)

        def outer(k, carry):
            step(2 * k, 0)
            step(2 * k + 1, 1)
            return carry

        lax.fori_loop(0, (nch + 1) // 2, outer, 0)
        pltpu.make_async_copy(yb[0], agg_sp.at[idx[0]], ssm[0]).wait()
        pltpu.make_async_copy(yb[1], agg_sp.at[idx[1]], ssm[1]).wait()
        plsc.subcore_barrier()
        for r in range(NPW // RB):
            row0 = sid * NPW + r * RB
            pltpu.sync_copy(agg_sp.at[pl.ds(row0, RB)], tb)
            pltpu.sync_copy(tb, agg_hbm.at[pl.ds(cid * NP + row0, RB)])

    return pl.kernel(
        body,
        out_type=jax.ShapeDtypeStruct((2 * NP, CP), F32),
        mesh=_mesh(),
        scratch_types=[
            pltpu.VMEM((KS,), jnp.int32), pltpu.VMEM((KS,), jnp.int32),
            pltpu.VMEM((KS, CP), F32), pltpu.VMEM((KS, CP), F32),
            pltpu.VMEM((RB, CP), F32), pltpu.VMEM((2, CP), F32),
            pltpu.VMEM_SHARED((NP, CP), F32),
            pltpu.SemaphoreType.DMA, pltpu.SemaphoreType.DMA,
            pltpu.SemaphoreType.DMA, pltpu.SemaphoreType.DMA,
        ],
    )(y2, dst, ab)


T_EDGE = 2000


@jax.jit
def _edge_matmul(y1, stats, g1, b1, W2, g2, b2):
    grid = E // T_EDGE

    def body(y_ref, st_ref, g1_ref, b1_ref, w_ref, g2_ref, b2_ref,
             y2_ref, ab_ref, acc):
        i = pl.program_id(0)
        st = st_ref[...].reshape(NW, 8, CP)
        ssum = jnp.sum(st[:, 0, :], axis=0)
        ssq = jnp.sum(st[:, 1, :], axis=0)
        mu = ssum / E
        var = ssq / E - mu * mu
        a1 = g1_ref[...] / jnp.sqrt(var[None, :] + EPS)
        b1p = b1_ref[...] - mu[None, :] * a1
        h = jnp.maximum(y_ref[...] * a1 + b1p, 0.0)
        y2 = jnp.dot(h, w_ref[...], preferred_element_type=F32,
                    precision=lax.Precision.HIGHEST)
        y2_ref[...] = y2
        part = jnp.stack([jnp.sum(y2, axis=0), jnp.sum(y2 * y2, axis=0)])
        acc[...] = jnp.where(i == 0, part, acc[...] + part)

        @pl.when(i == grid - 1)
        def _():
            mu2 = acc[0, :] / E
            var2 = acc[1, :] / E - mu2 * mu2
            a2 = g2_ref[0, :] / jnp.sqrt(var2 + EPS)
            b2p = b2_ref[0, :] - mu2 * a2
            ab_ref[...] = jnp.stack([a2, b2p])

    return pl.pallas_call(
        body,
        grid=(grid,),
        in_specs=[
            pl.BlockSpec((T_EDGE, CP), lambda i: (i, 0)),
            pl.BlockSpec((8 * NW, CP), lambda i: (0, 0)),
            pl.BlockSpec((1, CP), lambda i: (0, 0)),
            pl.BlockSpec((1, CP), lambda i: (0, 0)),
            pl.BlockSpec((CP, CP), lambda i: (0, 0)),
            pl.BlockSpec((1, CP), lambda i: (0, 0)),
            pl.BlockSpec((1, CP), lambda i: (0, 0)),
        ],
        out_specs=[
            pl.BlockSpec((T_EDGE, CP), lambda i: (i, 0)),
            pl.BlockSpec((2, CP), lambda i: (0, 0)),
        ],
        out_shape=[jax.ShapeDtypeStruct((E, CP), F32),
                   jax.ShapeDtypeStruct((2, CP), F32)],
        scratch_shapes=[pltpu.VMEM((2, CP), F32)],
    )(y1, stats, g1, b1, W2, g2, b2)


def _bn_cols(t, g, b):
    mu = jnp.mean(t, axis=0, keepdims=True)
    var = jnp.mean((t - mu) ** 2, axis=0, keepdims=True)
    return (t - mu) / jnp.sqrt(var + EPS) * g + b


@jax.jit
def _init_node(features, g, b):
    def body(f_ref, g_ref, b_ref, x_ref):
        x_ref[...] = _bn_cols(f_ref[...], g_ref[...], b_ref[...])

    d = features.shape[1]
    return pl.pallas_call(
        body,
        out_shape=jax.ShapeDtypeStruct((N, d), F32),
    )(features, g, b)


T_PQ = 2000


@jax.jit
def _pq(x, A, B):
    c = x.shape[1]

    def body(x_ref, a_ref, b_ref, p_ref, q_ref):
        xv = x_ref[...]
        u = jnp.dot(xv, a_ref[...], preferred_element_type=F32,
                    precision=lax.Precision.HIGHEST)
        v = jnp.dot(xv, b_ref[...], preferred_element_type=F32,
                    precision=lax.Precision.HIGHEST)
        p_ref[...] = u - v
        q_ref[...] = v

    return pl.pallas_call(
        body,
        grid=(N // T_PQ,),
        in_specs=[
            pl.BlockSpec((T_PQ, c), lambda i: (i, 0)),
            pl.BlockSpec((c, CP), lambda i: (0, 0)),
            pl.BlockSpec((c, CP), lambda i: (0, 0)),
        ],
        out_specs=[
            pl.BlockSpec((T_PQ, CP), lambda i: (i, 0)),
            pl.BlockSpec((T_PQ, CP), lambda i: (i, 0)),
        ],
        out_shape=[jax.ShapeDtypeStruct((N, CP), F32),
                   jax.ShapeDtypeStruct((N, CP), F32)],
    )(x, A, B)


@functools.partial(jax.jit, static_argnames=("c", "has_sc", "first"))
def _node_update(aggpair, degcol, x, scW, scg, scb, c, has_sc,
                 first):
    def body(*refs):
        it = iter(refs)
        ap_ref = next(it)
        deg_ref = None if first else next(it)
        x_ref = next(it)
        if has_sc:
            scw_ref, scg_ref, scb_ref = next(it), next(it), next(it)
        xo_ref = next(it)
        if first:
            dego_ref = next(it)
        ap = ap_ref[...]
        agg = ap[0, :N, :c] + ap[1, :N, :c]
        if first:
            deg = jnp.maximum(ap[0, :N, c:c + 1] + ap[1, :N, c:c + 1], 1.0)
            dego_ref[...] = deg
        else:
            deg = deg_ref[...]
        agg = agg / deg
        x = x_ref[...]
        if has_sc:
            sc = _bn_cols(
                jnp.dot(x, scw_ref[...], preferred_element_type=F32,
                    precision=lax.Precision.HIGHEST),
                scg_ref[...], scb_ref[...])[:, :c]
        else:
            sc = x
        xn = jnp.maximum(agg + sc, 0.0)
        xo_ref[...] = xn

    args = [aggpair]
    if not first:
        args.append(degcol)
    args.append(x)
    if has_sc:
        args += [scW, scg, scb]
    outs = [jax.ShapeDtypeStruct((N, c), F32)]
    if first:
        outs.append(jax.ShapeDtypeStruct((N, 1), F32))
    return pl.pallas_call(body, out_shape=outs)(*args)


@jax.jit
def _head_fuse(xs, fws, fus_bias):
    nx = len(xs)
    FO = fus_bias.shape[1]
    grid = N // T_PQ

    def body(*refs):
        x_refs = refs[:nx]
        fw_refs = refs[nx:2 * nx]
        fbias_ref = refs[2 * nx]
        h_ref, st_ref, acc = refs[2 * nx + 1:]
        i = pl.program_id(0)
        a = fbias_ref[...]
        for xr, wr in zip(x_refs, fw_refs):
            a = a + jnp.dot(xr[...], wr[...], preferred_element_type=F32,
                            precision=lax.Precision.HIGHEST)
        h = jnp.maximum(a, 0.0)
        h_ref[...] = h
        part = jnp.stack([jnp.sum(h, axis=0), jnp.sum(h * h, axis=0)])
        acc[...] = jnp.where(i == 0, part, acc[...] + part)

        @pl.when(i == grid - 1)
        def _():
            st_ref[...] = acc[...]

    cs = [x.shape[1] for x in xs]
    return pl.pallas_call(
        body,
        grid=(grid,),
        in_specs=(
            [pl.BlockSpec((T_PQ, c), lambda i: (i, 0)) for c in cs]
            + [pl.BlockSpec((c, FO), lambda i: (0, 0)) for c in cs]
            + [pl.BlockSpec((1, FO), lambda i: (0, 0))]),
        out_specs=[
            pl.BlockSpec((T_PQ, FO), lambda i: (i, 0)),
            pl.BlockSpec((2, FO), lambda i: (0, 0)),
        ],
        out_shape=[jax.ShapeDtypeStruct((N, FO), F32),
                   jax.ShapeDtypeStruct((2, FO), F32)],
        scratch_shapes=[pltpu.VMEM((2, FO), F32)],
    )(*xs, *fws, fus_bias)


@functools.partial(jax.jit, static_argnames=("nclass_in",))
def _head_out(hraw, st, gids, fus_g, fus_b, fc1_W, fc1_b, fc2_W, fc2_b,
              nclass_in):
    def body(h_ref, st_ref, gid_ref, fg_ref, fb_ref, w1_ref, b1_ref,
             w2_ref, b2_ref, out_ref):
        stv = st_ref[...]
        mu = stv[0:1, :] / N
        var = stv[1:2, :] / N - mu * mu
        ga = fg_ref[...] / jnp.sqrt(var + EPS)
        gb = fb_ref[...] - mu * ga
        gid = gid_ref[...]
        onehot = (gid == lax.broadcasted_iota(jnp.int32, (1, 64), 1)
                  ).astype(F32)
        gsum = lax.dot_general(onehot, h_ref[...], (((0,), (0,)), ((), ())),
                               preferred_element_type=F32,
                               precision=lax.Precision.HIGHEST)
        gcnt = jnp.maximum(jnp.sum(onehot, axis=0), 1.0).reshape(64, 1)
        gx = (gsum / gcnt) * ga + gb
        h1 = jnp.maximum(
            jnp.dot(gx, w1_ref[...], preferred_element_type=F32,
                    precision=lax.Precision.HIGHEST)
            + b1_ref[...], 0.0)
        out_ref[...] = (jnp.dot(h1, w2_ref[...], preferred_element_type=F32,
                    precision=lax.Precision.HIGHEST)
                        + b2_ref[...])[:, :out_ref.shape[1]]

    return pl.pallas_call(
        body,
        out_shape=jax.ShapeDtypeStruct((64, nclass_in), F32),
    )(hraw, st, gids, fus_g, fus_b, fc1_W, fc1_b, fc2_W, fc2_b)


def kernel(features, edge_index, graph_ids, params):
    src = edge_index[0]
    dst = edge_index[1]
    blocks = params["blocks"]
    nblk = len(blocks)
    AB = []
    feats = [features.shape[1]]
    for blk in blocks:
        f = feats[-1]
        W1 = blk["convs"][0]["W"]
        AB.append((_pad_cols(W1[:f]), _pad_cols(W1[f:])))
        feats.append(blk["convs"][-1]["W"].shape[1])

    x = _init_node(features, params["bn0_g"].reshape(1, -1),
                   params["bn0_b"].reshape(1, -1))
    P, Q = _pq(x, AB[0][0], AB[0][1])
    degcol = None
    outputs = []
    for bi, blk in enumerate(blocks):
        c = feats[bi + 1]
        cv1, cv2 = blk["convs"]
        y1, stats = _edge_gather(P, Q, src, dst, c=c)
        y2, ab = _edge_matmul(
            y1, stats, _pad_vec(cv1["g"]), _pad_vec(cv1["b"]),
            _pad_mat(cv2["W"]), _pad_vec(cv2["g"]), _pad_vec(cv2["b"]))
        first = bi == 0
        aggflat = _edge_scatter(y2, dst, ab, c=c, with_ones=first)
        aggpair = aggflat.reshape(2, NP, CP)
        has_sc = "sc_W" in blk
        res = _node_update(
            aggpair, degcol, x,
            _pad_cols(blk["sc_W"]) if has_sc else None,
            _pad_vec(blk["sc_g"]) if has_sc else None,
            _pad_vec(blk["sc_b"]) if has_sc else None,
            c=c, has_sc=has_sc, first=first)
        x = res[0]
        if first:
            degcol = res[-1]
        if bi + 1 < nblk:
            P, Q = _pq(x, AB[bi + 1][0], AB[bi + 1][1])
        outputs.append(x)

    fw = params["fus_W"]
    fws, off = [], 0
    for bi in range(nblk):
        fws.append(fw[off:off + feats[bi + 1]])
        off += feats[bi + 1]
    hraw, st = _head_fuse(outputs, fws, params["fus_bias"].reshape(1, -1))
    return _head_out(hraw, st, graph_ids.reshape(N, 1).astype(jnp.int32),
                     params["fus_g"].reshape(1, -1),
                     params["fus_b"].reshape(1, -1),
                     params["fc1_W"], params["fc1_b"].reshape(1, -1),
                     _pad_cols(params["fc2_W"]), _pad_vec(params["fc2_b"]),
                     nclass_in=params["fc2_W"].shape[1])
